# baseline (speedup 1.0000x reference)
import functools

import jax
import jax.numpy as jnp
from jax import lax
from jax.experimental import pallas as pl
from jax.experimental.pallas import tpu as pltpu
from jax.experimental.pallas import tpu_sc as plsc

_ROWS = 16384
_COLS = 200
_B = _ROWS * _COLS
_NC = 2
_NS = 16
_NW = _NC * _NS
_PW = _B // _NW
_NCHUNK = 8
_C = _PW // _NCHUNK
_V = 1000000
_STAGE = 62504


def _make_gather():
    mesh = plsc.VectorSubcoreMesh(core_axis_name="c", subcore_axis_name="s")

    @functools.partial(
        pl.kernel,
        out_type=jax.ShapeDtypeStruct((_B,), jnp.float32),
        mesh=mesh,
        scratch_types=[
            pltpu.VMEM((_C,), jnp.int32),
            pltpu.VMEM((_C,), jnp.int32),
            pltpu.VMEM((_C,), jnp.float32),
            pltpu.VMEM((_C,), jnp.float32),
            pltpu.SemaphoreType.DMA,
            pltpu.SemaphoreType.DMA,
            pltpu.SemaphoreType.DMA,
            pltpu.SemaphoreType.DMA,
            pltpu.SemaphoreType.DMA,
            pltpu.SemaphoreType.DMA,
            pltpu.VMEM_SHARED((_V,), jnp.float32),
        ],
    )
    def gather_kernel(x_hbm, idx_hbm, out_hbm,
                      i0, i1, r0, r1, si0, si1, sg0, sg1, so0, so1,
                      table_sh):
        idx_bufs, row_bufs = (i0, i1), (r0, r1)
        si, sg, so = (si0, si1), (sg0, sg1), (so0, so1)
        sid = lax.axis_index("s")
        wid = sid * _NC + lax.axis_index("c")
        base = wid * _PW

        stage_off = sid * _STAGE

        def stage(off, size):
            pltpu.sync_copy(x_hbm.at[pl.ds(off, size)], r0.at[pl.ds(0, size)])
            pltpu.sync_copy(r0.at[pl.ds(0, size)], table_sh.at[pl.ds(off, size)])

        _SCH = 12504
        for k in range(4):
            stage(stage_off + k * _SCH, _SCH)

        @pl.when(sid < _NS - 1)
        def _():
            stage(stage_off + 4 * _SCH, _STAGE - 4 * _SCH)

        @pl.when(sid == _NS - 1)
        def _():
            tail = _V - (_NS - 1) * _STAGE
            stage(stage_off + 4 * _SCH, tail - 4 * _SCH)

        plsc.subcore_barrier()

        def idx_cp(c):
            b = c % 2
            return pltpu.make_async_copy(
                idx_hbm.at[pl.ds(base + c * _C, _C)], idx_bufs[b], si[b])

        srcs = (table_sh, x_hbm, table_sh, x_hbm, table_sh, x_hbm,
                table_sh, table_sh)

        def gather_cp(c):
            b = c % 2
            return pltpu.make_async_copy(
                srcs[c].at[idx_bufs[b]], row_bufs[b], sg[b])

        def out_cp(c):
            b = c % 2
            return pltpu.make_async_copy(
                row_bufs[b], out_hbm.at[pl.ds(base + c * _C, _C)], so[b])

        idx_cp(0).start()
        idx_cp(1).start()
        for c in range(_NCHUNK):
            idx_cp(c).wait()
            if c >= 2:
                out_cp(c - 2).wait()
            gather_cp(c).start()
            if c >= 1:
                gather_cp(c - 1).wait()
                out_cp(c - 1).start()
                if c + 1 < _NCHUNK:
                    idx_cp(c + 1).start()
        gather_cp(_NCHUNK - 1).wait()
        out_cp(_NCHUNK - 1).start()
        out_cp(_NCHUNK - 2).wait()
        out_cp(_NCHUNK - 1).wait()

    return gather_kernel


_gather = _make_gather()


@jax.jit
def kernel(x, index):
    idx_flat = index.reshape(-1).astype(jnp.int32)
    out = _gather(x, idx_flat)
    return out.reshape(index.shape)


# speedup vs baseline: 1.1992x; 1.1992x over previous
import functools

import jax
import jax.numpy as jnp
from jax import lax
from jax.experimental import pallas as pl
from jax.experimental.pallas import tpu as pltpu
from jax.experimental.pallas import tpu_sc as plsc

_ROWS = 16384
_COLS = 200
_B = _ROWS * _COLS
_NC = 2
_NS = 16
_NW = _NC * _NS
_PW = _B // _NW
_NCHUNK = 8
_C = _PW // _NCHUNK
_V = 1000000
_STAGE = 62504
_SCH = 7816
_NST = 7


def _make_gather():
    mesh = plsc.VectorSubcoreMesh(core_axis_name="c", subcore_axis_name="s")

    @functools.partial(
        pl.kernel,
        out_type=jax.ShapeDtypeStruct((_B,), jnp.float32),
        mesh=mesh,
        scratch_types=[
            pltpu.VMEM((_C,), jnp.int32),
            pltpu.VMEM((_C,), jnp.int32),
            pltpu.VMEM((_C,), jnp.float32),
            pltpu.VMEM((_C,), jnp.float32),
            pltpu.VMEM((_SCH,), jnp.float32),
            pltpu.VMEM((_SCH,), jnp.float32),
            pltpu.SemaphoreType.DMA,
            pltpu.SemaphoreType.DMA,
            pltpu.SemaphoreType.DMA,
            pltpu.SemaphoreType.DMA,
            pltpu.SemaphoreType.DMA,
            pltpu.SemaphoreType.DMA,
            pltpu.SemaphoreType.DMA,
            pltpu.SemaphoreType.DMA,
            pltpu.SemaphoreType.DMA,
            pltpu.SemaphoreType.DMA,
            pltpu.VMEM_SHARED((_V,), jnp.float32),
        ],
    )
    def gather_kernel(x_hbm, idx_hbm, out_hbm,
                      i0, i1, r0, r1, sb0, sb1,
                      si0, si1, sg0, sg1, so0, so1, ss0, ss1, st0, st1,
                      table_sh):
        idx_bufs, row_bufs, st_bufs = (i0, i1), (r0, r1), (sb0, sb1)
        si, sg, so = (si0, si1), (sg0, sg1), (so0, so1)
        ss, st = (ss0, ss1), (st0, st1)
        sid = lax.axis_index("s")
        wid = sid * _NC + lax.axis_index("c")
        base = wid * _PW
        stage_off = sid * _STAGE

        def idx_cp(c):
            b = c % 2
            return pltpu.make_async_copy(
                idx_hbm.at[pl.ds(base + c * _C, _C)], idx_bufs[b], si[b])

        def gather_cp(c, src):
            b = c % 2
            return pltpu.make_async_copy(
                src.at[idx_bufs[b]], row_bufs[b], sg[b])

        def out_cp(c):
            b = c % 2
            return pltpu.make_async_copy(
                row_bufs[b], out_hbm.at[pl.ds(base + c * _C, _C)], so[b])

        def hop1(k, size):
            b = k % 2
            return pltpu.make_async_copy(
                x_hbm.at[pl.ds(stage_off + k * _SCH, size)],
                st_bufs[b].at[pl.ds(0, size)], ss[b])

        def hop2(k, size):
            b = k % 2
            return pltpu.make_async_copy(
                st_bufs[b].at[pl.ds(0, size)],
                table_sh.at[pl.ds(stage_off + k * _SCH, size)], st[b])

        hop1(0, _SCH).start()
        hop1(1, _SCH).start()

        for k in range(_NST):
            hop1(k, _SCH).wait()
            hop2(k, _SCH).start()
            hop2(k, _SCH).wait()
            if k + 2 < _NST:
                hop1(k + 2, _SCH).start()

        @pl.when(sid < _NS - 1)
        def _():
            sz = _STAGE - _NST * _SCH
            hop1(_NST, sz).start()
            hop1(_NST, sz).wait()
            hop2(_NST, sz).start()
            hop2(_NST, sz).wait()

        @pl.when(sid == _NS - 1)
        def _():
            sz = (_V - (_NS - 1) * _STAGE) - _NST * _SCH
            hop1(_NST, sz).start()
            hop1(_NST, sz).wait()
            hop2(_NST, sz).start()
            hop2(_NST, sz).wait()

        plsc.subcore_barrier()

        idx_cp(0).start()
        idx_cp(1).start()
        idx_cp(0).wait()
        gather_cp(0, table_sh).start()
        for c in range(1, _NCHUNK):
            idx_cp(c).wait()
            if c >= 2:
                out_cp(c - 2).wait()
            gather_cp(c, table_sh).start()
            gather_cp(c - 1, table_sh).wait()
            out_cp(c - 1).start()
            if c + 1 < _NCHUNK:
                idx_cp(c + 1).start()
        gather_cp(_NCHUNK - 1, table_sh).wait()
        out_cp(_NCHUNK - 1).start()
        out_cp(_NCHUNK - 2).wait()
        out_cp(_NCHUNK - 1).wait()

    return gather_kernel


_gather = _make_gather()


@jax.jit
def kernel(x, index):
    idx_flat = index.reshape(-1).astype(jnp.int32)
    out = _gather(x, idx_flat)
    return out.reshape(index.shape)


# speedup vs baseline: 1.3124x; 1.0944x over previous
import functools

import jax
import jax.numpy as jnp
from jax import lax
from jax.experimental import pallas as pl
from jax.experimental.pallas import tpu as pltpu
from jax.experimental.pallas import tpu_sc as plsc

_ROWS = 16384
_COLS = 200
_TAIL0 = 184
_NC = 2
_NS = 16
_NW = _NC * _NS
_RPW = _ROWS // _NW
_NCHUNK = 16
_RC = _RPW // _NCHUNK
_CW = _RC * _COLS
_V = 1000000
_STAGE = 62504
_SCH = 7032
_NST = 8
_OFFS = tuple(range(0, _TAIL0, 16))


def _make_gather():
    mesh = plsc.VectorSubcoreMesh(core_axis_name="c", subcore_axis_name="s")

    @functools.partial(
        pl.kernel,
        out_type=jax.ShapeDtypeStruct((_ROWS, _COLS), jnp.float32),
        mesh=mesh,
        compiler_params=pltpu.CompilerParams(use_tc_tiling_on_sc=True),
        scratch_types=[
            pltpu.VMEM((_RC, _COLS), jnp.int32),
            pltpu.VMEM((_RC, 16), jnp.int32),
            pltpu.VMEM((_RC, _COLS), jnp.float32),
            pltpu.VMEM((_RC, _COLS), jnp.float32),
            pltpu.VMEM((_CW,), jnp.int32),
            pltpu.VMEM((_CW,), jnp.int32),
            pltpu.VMEM((_CW,), jnp.float32),
            pltpu.VMEM((_CW,), jnp.float32),
            pltpu.VMEM((_SCH,), jnp.float32),
            pltpu.VMEM((_SCH,), jnp.float32),
            pltpu.SemaphoreType.DMA,
            pltpu.SemaphoreType.DMA,
            pltpu.SemaphoreType.DMA,
            pltpu.SemaphoreType.DMA,
            pltpu.SemaphoreType.DMA,
            pltpu.SemaphoreType.DMA,
            pltpu.SemaphoreType.DMA,
            pltpu.SemaphoreType.DMA,
            pltpu.SemaphoreType.DMA,
            pltpu.VMEM_SHARED((_V,), jnp.float32),
        ],
    )
    def gather_kernel(x_hbm, idx_hbm, tail_hbm, out_hbm,
                      iv_t, tv_t, rv_t0, rv_t1, il0, il1, rl0, rl1,
                      sb0, sb1,
                      si, sg0, sg1, so0, so1, ss0, ss1, st0, st1,
                      table_sh):
        rv_t = (rv_t0, rv_t1)
        idx_lin, row_lin, st_bufs = (il0, il1), (rl0, rl1), (sb0, sb1)
        sg, so, ss, st = (sg0, sg1), (so0, so1), (ss0, ss1), (st0, st1)
        sid = lax.axis_index("s")
        wid = sid * _NC + lax.axis_index("c")
        base_row = wid * _RPW
        stage_off = sid * _STAGE

        def idx_t_cp(c):
            return pltpu.make_async_copy(
                idx_hbm.at[pl.ds(base_row + c * _RC, _RC), :], iv_t, si)

        def tail_t_cp(c):
            return pltpu.make_async_copy(
                tail_hbm.at[pl.ds(base_row + c * _RC, _RC), :], tv_t, si)

        def convert_idx(c):
            b = c % 2

            def row(r, carry):
                for off in _OFFS:
                    v = iv_t[r, pl.ds(off, 16)]
                    idx_lin[b][pl.ds(r * _COLS + off, 16)] = \
                        jnp.minimum(jnp.maximum(v, 0), _V - 1)
                vt = tv_t[r, pl.ds(0, 16)]
                idx_lin[b][pl.ds(r * _COLS + _TAIL0, 16)] = \
                    jnp.minimum(jnp.maximum(vt, 0), _V - 1)
                return carry

            lax.fori_loop(0, _RC, row, 0)

        def gather_cp(c, src):
            b = c % 2
            return pltpu.make_async_copy(
                src.at[idx_lin[b]], row_lin[b], sg[b])

        def convert_rows(c):
            b = c % 2

            def row(r, carry):
                for off in _OFFS:
                    rv_t[b][r, pl.ds(off, 16)] = \
                        row_lin[b][pl.ds(r * _COLS + off, 16)]
                pltpu.make_async_copy(
                    table_sh.at[idx_lin[b].at[pl.ds(r * _COLS + 192, 8)]],
                    rv_t[b].at[r, pl.ds(192, 8)], ss[b]).start()
                return carry

            lax.fori_loop(0, _RC, row, 0)

        def tail_drain(c):
            b = c % 2
            pltpu.make_async_copy(
                x_hbm.at[pl.ds(0, 8 * _RC)],
                st_bufs[0].at[pl.ds(0, 8 * _RC)], ss[b]).wait()

        def out_t_cp(c):
            b = c % 2
            return pltpu.make_async_copy(
                rv_t[b], out_hbm.at[pl.ds(base_row + c * _RC, _RC), :],
                so[b])

        def hop1(k, size):
            b = k % 2
            return pltpu.make_async_copy(
                x_hbm.at[pl.ds(stage_off + k * _SCH, size)],
                st_bufs[b].at[pl.ds(0, size)], ss[b])

        def hop2(k, size):
            b = k % 2
            return pltpu.make_async_copy(
                st_bufs[b].at[pl.ds(0, size)],
                table_sh.at[pl.ds(stage_off + k * _SCH, size)], st[b])

        hop1(0, _SCH).start()
        hop1(1, _SCH).start()

        for k in range(_NST):
            hop1(k, _SCH).wait()
            hop2(k, _SCH).start()
            hop2(k, _SCH).wait()
            if k + 2 < _NST:
                hop1(k + 2, _SCH).start()

        @pl.when(sid < _NS - 1)
        def _():
            sz = _STAGE - _NST * _SCH
            hop1(_NST, sz).start()
            hop1(_NST, sz).wait()
            hop2(_NST, sz).start()
            hop2(_NST, sz).wait()

        @pl.when(sid == _NS - 1)
        def _():
            sz = (_V - (_NS - 1) * _STAGE) - _NST * _SCH
            hop1(_NST, sz).start()
            hop1(_NST, sz).wait()
            hop2(_NST, sz).start()
            hop2(_NST, sz).wait()

        plsc.subcore_barrier()

        idx_t_cp(0).start()
        tail_t_cp(0).start()
        idx_t_cp(0).wait()
        tail_t_cp(0).wait()
        convert_idx(0)
        idx_t_cp(1).start()
        tail_t_cp(1).start()
        gather_cp(0, table_sh).start()
        idx_t_cp(1).wait()
        tail_t_cp(1).wait()
        convert_idx(1)
        idx_t_cp(2).start()
        tail_t_cp(2).start()
        for c in range(1, _NCHUNK):
            gather_cp(c, table_sh).start()
            gather_cp(c - 1, table_sh).wait()
            if c >= 3:
                out_t_cp(c - 3).wait()
            convert_rows(c - 1)
            tail_drain(c - 1)
            out_t_cp(c - 1).start()
            if c + 1 < _NCHUNK:
                idx_t_cp(c + 1).wait()
                tail_t_cp(c + 1).wait()
                convert_idx(c + 1)
                if c + 2 < _NCHUNK:
                    idx_t_cp(c + 2).start()
                    tail_t_cp(c + 2).start()
        gather_cp(_NCHUNK - 1, table_sh).wait()
        out_t_cp(_NCHUNK - 3).wait()
        convert_rows(_NCHUNK - 1)
        tail_drain(_NCHUNK - 1)
        out_t_cp(_NCHUNK - 1).start()
        out_t_cp(_NCHUNK - 2).wait()
        out_t_cp(_NCHUNK - 1).wait()

    return gather_kernel


_gather = _make_gather()


@jax.jit
def kernel(x, index):
    idx = index.astype(jnp.int32)
    return _gather(x, idx, idx[:, _TAIL0:_COLS])
